# baseline (speedup 1.0000x reference)
import jax
import jax.numpy as jnp
from jax import lax
from jax.experimental import pallas as pl
from jax.experimental.pallas import tpu as pltpu
from jax.experimental.pallas import tpu_sc as plsc

_B, _N, _BINS = 16, 65536, 32
_K = _BINS ** 3
_NC, _NS, _L = 2, 16, 16
_HALF = _N // 2
_CH = 1024
_NCH = _HALF // _CH
_VPC = _CH // _L
_PPC = _VPC // 2
_GRP = 512
_NG = _CH // _GRP
_KW = _K // 2
_HW = _K // 2
_ILV = plsc.PackFormat.INTERLEAVED


def _sort_body(px_hbm, py_hbm, pz_hbm, curve_hbm, par_hbm,
               op_hbm, oi_hbm,
               cpk_v, keys_v, hist_v,
               xv, yv, zv, eidx_v, ival_v, par_v,
               perm_sh, semA, semB, semGA, semGB, semI):
  c = lax.axis_index("c")
  s = lax.axis_index("s")
  batch = c * (_B // _NC) + s // 2
  pair = s // 2
  half = s % 2
  pt_base = batch * _N + half * _HALF
  iota = lax.iota(jnp.int32, _L)
  planes = ((px_hbm, xv), (py_hbm, yv), (pz_hbm, zv))
  gsem = (semGA, semGB)
  psem = (semA, semB)

  pltpu.sync_copy(par_hbm, par_v)
  pltpu.sync_copy(curve_hbm, cpk_v)
  ox = par_v[0, :]
  oy = par_v[1, :]
  oz = par_v[2, :]
  itv = par_v[3, :]
  zero16 = jnp.zeros((_L,), jnp.int32)
  c16 = jnp.full((_L,), 16.0, jnp.float32)
  lo = jnp.zeros((_L,), jnp.int32)
  hi = jnp.full((_L,), _BINS - 1, jnp.int32)
  m16 = jnp.full((_L,), 0xFFFF, jnp.int32)
  s16 = jnp.full((_L,), 16, jnp.int32)
  one = jnp.full((_L,), 1, jnp.int32)

  def zero_hist(i, _):
    hist_v[pl.ds(i * _L, _L)] = zero16
    return 0
  lax.fori_loop(0, _K // _L, zero_hist, 0)

  def p1_load(ch, k):
    p0 = pt_base + ch * _CH
    for hbm, buf in planes:
      pltpu.async_copy(hbm.at[pl.ds(p0, _CH)], buf.at[pl.ds(k * _CH, _CH)],
                       psem[k])

  def p1_drain(k):
    for hbm, buf in planes:
      pltpu.make_async_copy(hbm.at[pl.ds(0, _CH)],
                            buf.at[pl.ds(k * _CH, _CH)], psem[k]).wait()

  def key_vreg(v, k):
    x = xv[pl.ds(k * _CH + v * _L, _L)]
    y = yv[pl.ds(k * _CH + v * _L, _L)]
    z = zv[pl.ds(k * _CH + v * _L, _L)]
    bx = jnp.clip(((x - ox) / itv + c16).astype(jnp.int32), lo, hi)
    by = jnp.clip(((y - oy) / itv + c16).astype(jnp.int32), lo, hi)
    bz = jnp.clip(((z - oz) / itv + c16).astype(jnp.int32), lo, hi)
    lin = (bx * _BINS + by) * _BINS + bz
    w = plsc.load_gather(cpk_v, [lax.shift_right_logical(lin, one)])
    return jnp.where(lin & one == one,
                     lax.shift_right_logical(w, s16), w & m16)

  def hist_add(hk):
    cnt, last = plsc.scan_count(hk)
    plsc.addupdate_scatter(hist_v, [hk], cnt, mask=last)

  def p1_compute(ch, k):
    def p1_pair(u, _):
      hk0 = key_vreg(2 * u, k)
      hk1 = key_vreg(2 * u + 1, k)
      keys_v[pl.ds(ch * _CH + u * 32, 32)] = plsc.pack(
          hk0, hk1, format=_ILV, preferred_element_type=jnp.int16)
      hist_add(hk0)
      hist_add(hk1)
      return 0
    lax.fori_loop(0, _PPC, p1_pair, 0)

  p1_load(0, 0)
  def p1_iter(i, _):
    ch0 = 2 * i
    p1_drain(0)
    p1_load(ch0 + 1, 1)
    p1_compute(ch0, 0)
    p1_drain(1)
    p1_load((ch0 + 2) % _NCH, 0)
    p1_compute(ch0 + 1, 1)
    return 0
  lax.fori_loop(0, _NCH // 2, p1_iter, 0)
  p1_drain(0)

  def pub_pair(i, _):
    m0 = hist_v[pl.ds(i * 32, _L)]
    m1 = hist_v[pl.ds(i * 32 + _L, _L)]
    cpk_v[pl.ds(i * _L, _L)] = plsc.bitcast(
        plsc.pack(m0, m1, format=_ILV, preferred_element_type=jnp.int16),
        jnp.int32)
    return 0
  lax.fori_loop(0, _HW // _L, pub_pair, 0)
  pltpu.sync_copy(cpk_v, perm_sh.at[pl.ds(s * _HW, _HW)])
  plsc.subcore_barrier()
  partner = s + 1 - 2 * half
  pltpu.sync_copy(perm_sh.at[pl.ds(partner * _HW, _HW)], cpk_v)
  hb = jnp.full((_L,), half, jnp.int32)

  def p2_pair(i, carry):
    p0, p1 = plsc.unpack(
        plsc.bitcast(cpk_v[pl.ds(i * _L, _L)], jnp.int16),
        format=_ILV, preferred_element_type=jnp.int32)
    p0 = p0 & m16
    p1 = p1 & m16
    m0 = hist_v[pl.ds(i * 32, _L)]
    m1 = hist_v[pl.ds(i * 32 + _L, _L)]
    tot0 = m0 + p0
    tot1 = m1 + p1
    incl0 = plsc.cumsum(tot0)
    incl1 = plsc.cumsum(tot1)
    carry1 = carry + jnp.sum(tot0)
    off0 = incl0 - tot0 + jnp.full((_L,), carry, jnp.int32) + p0 * hb
    off1 = incl1 - tot1 + jnp.full((_L,), carry1, jnp.int32) + p1 * hb
    hist_v[pl.ds(i * 32, _L)] = off0
    hist_v[pl.ds(i * 32 + _L, _L)] = off1
    return carry1 + jnp.sum(tot1)
  lax.fori_loop(0, _K // 32, p2_pair, jnp.int32(0))
  plsc.subcore_barrier()

  stage_base = jnp.full((_L,), pair * _N, jnp.int32)

  def rank_vreg(hk):
    cnt, last = plsc.scan_count(hk)
    base = plsc.load_gather(hist_v, [hk])
    plsc.addupdate_scatter(hist_v, [hk], cnt, mask=last)
    return base + cnt - one + stage_base

  def p3_compute(ch, k):
    def p3_pair(u, _):
      k0, k1 = plsc.unpack(
          keys_v[pl.ds(ch * _CH + u * 32, 32)],
          format=_ILV, preferred_element_type=jnp.int32)
      v = 2 * u
      pos0 = rank_vreg(k0)
      pos1 = rank_vreg(k1)
      g = v // (_GRP // _L)
      lane0 = (v % (_GRP // _L)) * _L
      eidx_v[k * _NG + g, pl.ds(lane0, _L)] = pos0
      eidx_v[k * _NG + g, pl.ds(lane0 + _L, _L)] = pos1
      ibase = half * _HALF + ch * _CH + v * _L
      ival_v[pl.ds(k * _CH + v * _L, _L)] = iota + jnp.full(
          (_L,), ibase, jnp.int32)
      ival_v[pl.ds(k * _CH + v * _L + _L, _L)] = iota + jnp.full(
          (_L,), ibase + _L, jnp.int32)
      return 0
    lax.fori_loop(0, _PPC, p3_pair, 0)

  def p3_fire(k):
    for g in range(_NG):
      pltpu.async_copy(ival_v.at[pl.ds(k * _CH + g * _GRP, _GRP)],
                       perm_sh.at[eidx_v.at[k * _NG + g]], gsem[k])

  def p3_drain(k):
    for g in range(_NG):
      pltpu.make_async_copy(ival_v.at[pl.ds(k * _CH + g * _GRP, _GRP)],
                            perm_sh.at[eidx_v.at[k * _NG + g]],
                            gsem[k]).wait()

  p3_compute(0, 0)
  p3_fire(0)
  p3_compute(1, 1)
  p3_fire(1)
  def p3_iter(i, _):
    ch0 = 2 * i
    p3_drain(0)
    p3_compute(ch0, 0)
    p3_fire(0)
    p3_drain(1)
    p3_compute(ch0 + 1, 1)
    p3_fire(1)
    return 0
  lax.fori_loop(1, _NCH // 2, p3_iter, 0)
  p3_drain(0)
  p3_drain(1)
  plsc.subcore_barrier()

  gbase = jnp.full((_L,), batch * _N, jnp.int32)
  gpr = _GRP // _L

  def p4_prep(ch, k):
    out0 = pt_base + ch * _CH
    stage0 = pair * _N + half * _HALF + ch * _CH
    pltpu.sync_copy(perm_sh.at[pl.ds(stage0, _CH)],
                    ival_v.at[pl.ds(k * _CH, _CH)])
    pltpu.async_copy(perm_sh.at[pl.ds(stage0, _CH)],
                     oi_hbm.at[pl.ds(out0, _CH)], semI)
    def p4_bld(v, _):
      g = v // gpr
      lane0 = (v % gpr) * _L
      eidx_v[k * _NG + g, pl.ds(lane0, _L)] = ival_v[
          pl.ds(k * _CH + v * _L, _L)] + gbase
      return 0
    lax.fori_loop(0, _VPC, p4_bld, 0)

  def p4_fire(k):
    for g in range(_NG):
      for hbm, buf in planes:
        pltpu.async_copy(hbm.at[eidx_v.at[k * _NG + g]],
                         buf.at[pl.ds(k * _CH + g * _GRP, _GRP)], gsem[k])

  def p4_drain(k):
    for g in range(_NG):
      for hbm, buf in planes:
        pltpu.make_async_copy(hbm.at[eidx_v.at[k * _NG + g]],
                              buf.at[pl.ds(k * _CH + g * _GRP, _GRP)],
                              gsem[k]).wait()

  def p4_write(ch, k):
    out0 = pt_base + ch * _CH
    for p, (hbm, buf) in enumerate(planes):
      pltpu.sync_copy(buf.at[pl.ds(k * _CH, _CH)],
                      op_hbm.at[pl.ds(p * _B * _N + out0, _CH)])

  p4_prep(0, 0)
  p4_fire(0)
  def p4_iter(i, _):
    ch0 = 2 * i
    p4_prep(ch0 + 1, 1)
    p4_fire(1)
    p4_drain(0)
    p4_write(ch0, 0)
    p4_prep((ch0 + 2) % _NCH, 0)
    p4_fire(0)
    p4_drain(1)
    p4_write(ch0 + 1, 1)
    return 0
  lax.fori_loop(0, _NCH // 2, p4_iter, 0)
  p4_drain(0)
  def p4_idx_drain(ch, _):
    pltpu.make_async_copy(perm_sh.at[pl.ds(pair * _N, _CH)],
                          oi_hbm.at[pl.ds(pt_base, _CH)], semI).wait()
    return 0
  lax.fori_loop(0, _NCH + 1, p4_idx_drain, 0)


@jax.jit
def kernel(point_cloud, origin, radius, curve):
  bins = curve.shape[0]
  bin_interval = radius * 2.0 / bins
  par = jnp.stack([
      jnp.full((_L,), origin[0], jnp.float32),
      jnp.full((_L,), origin[1], jnp.float32),
      jnp.full((_L,), origin[2], jnp.float32),
      jnp.full((_L,), bin_interval, jnp.float32),
  ])
  cv = curve.reshape(_KW, 2).astype(jnp.int32)
  curve_packed = cv[:, 0] | (cv[:, 1] << 16)
  px = point_cloud[:, :, 0].reshape(_B * _N)
  py = point_cloud[:, :, 1].reshape(_B * _N)
  pz = point_cloud[:, :, 2].reshape(_B * _N)

  mesh = plsc.VectorSubcoreMesh(
      core_axis_name="c", subcore_axis_name="s",
      num_cores=_NC, num_subcores=_NS)
  splanes, sorted_idx_flat = pl.kernel(
      _sort_body,
      out_type=[
          jax.ShapeDtypeStruct((3 * _B * _N,), jnp.float32),
          jax.ShapeDtypeStruct((_B * _N,), jnp.int32),
      ],
      mesh=mesh,
      scratch_types=[
          pltpu.VMEM((_KW,), jnp.int32),
          pltpu.VMEM((_HALF,), jnp.int16),
          pltpu.VMEM((_K,), jnp.int32),
          pltpu.VMEM((2 * _CH,), jnp.float32),
          pltpu.VMEM((2 * _CH,), jnp.float32),
          pltpu.VMEM((2 * _CH,), jnp.float32),
          pltpu.VMEM((2 * _NG, _GRP), jnp.int32),
          pltpu.VMEM((2 * _CH,), jnp.int32),
          pltpu.VMEM((4, _L), jnp.float32),
          pltpu.VMEM_SHARED((_NS * _K,), jnp.int32),
          pltpu.SemaphoreType.DMA,
          pltpu.SemaphoreType.DMA,
          pltpu.SemaphoreType.DMA,
          pltpu.SemaphoreType.DMA,
          pltpu.SemaphoreType.DMA,
      ],
      compiler_params=pltpu.CompilerParams(
          needs_layout_passes=False, use_tc_tiling_on_sc=False),
  )(px, py, pz, curve_packed, par)
  sorted_points = (splanes.reshape(3, _B, _N).transpose(1, 2, 0)
                   - origin[None, None, :])
  return (sorted_points, sorted_idx_flat.reshape(_B, _N))


# speedup vs baseline: 1.0091x; 1.0091x over previous
import jax
import jax.numpy as jnp
from jax import lax
from jax.experimental import pallas as pl
from jax.experimental.pallas import tpu as pltpu
from jax.experimental.pallas import tpu_sc as plsc

_B, _N, _BINS = 16, 65536, 32
_K = _BINS ** 3
_NC, _NS, _L = 2, 16, 16
_HALF = _N // 2
_CH = 1024
_NCH = _HALF // _CH
_VPC = _CH // _L
_PPC = _VPC // 2
_GRP = 128
_NG = _CH // _GRP
_KW = _K // 2
_HW = _K // 2
_ILV = plsc.PackFormat.INTERLEAVED


def _sort_body(px_hbm, py_hbm, pz_hbm, curve_hbm, par_hbm,
               op_hbm, oi_hbm,
               cpk_v, keys_v, hist_v,
               xv, yv, zv, eidx_v, ival_v, par_v,
               perm_sh, semA, semB, semGA, semGB):
  c = lax.axis_index("c")
  s = lax.axis_index("s")
  batch = c * (_B // _NC) + s // 2
  pair = s // 2
  half = s % 2
  pt_base = batch * _N + half * _HALF
  iota = lax.iota(jnp.int32, _L)
  planes = ((px_hbm, xv), (py_hbm, yv), (pz_hbm, zv))
  gsem = (semGA, semGB)
  psem = (semA, semB)

  pltpu.sync_copy(par_hbm, par_v)
  pltpu.sync_copy(curve_hbm, cpk_v)
  ox = par_v[0, :]
  oy = par_v[1, :]
  oz = par_v[2, :]
  itv = par_v[3, :]
  zero16 = jnp.zeros((_L,), jnp.int32)
  c16 = jnp.full((_L,), 16.0, jnp.float32)
  lo = jnp.zeros((_L,), jnp.int32)
  hi = jnp.full((_L,), _BINS - 1, jnp.int32)
  m16 = jnp.full((_L,), 0xFFFF, jnp.int32)
  s16 = jnp.full((_L,), 16, jnp.int32)
  one = jnp.full((_L,), 1, jnp.int32)

  def zero_hist(i, _):
    hist_v[pl.ds(i * _L, _L)] = zero16
    return 0
  lax.fori_loop(0, _K // _L, zero_hist, 0)

  def p1_load(ch, k):
    p0 = pt_base + ch * _CH
    for hbm, buf in planes:
      pltpu.async_copy(hbm.at[pl.ds(p0, _CH)], buf.at[pl.ds(k * _CH, _CH)],
                       psem[k])

  def p1_drain(k):
    for hbm, buf in planes:
      pltpu.make_async_copy(hbm.at[pl.ds(0, _CH)],
                            buf.at[pl.ds(k * _CH, _CH)], psem[k]).wait()

  def key_vreg(v, k):
    x = xv[pl.ds(k * _CH + v * _L, _L)]
    y = yv[pl.ds(k * _CH + v * _L, _L)]
    z = zv[pl.ds(k * _CH + v * _L, _L)]
    bx = jnp.clip(((x - ox) / itv + c16).astype(jnp.int32), lo, hi)
    by = jnp.clip(((y - oy) / itv + c16).astype(jnp.int32), lo, hi)
    bz = jnp.clip(((z - oz) / itv + c16).astype(jnp.int32), lo, hi)
    lin = (bx * _BINS + by) * _BINS + bz
    w = plsc.load_gather(cpk_v, [lax.shift_right_logical(lin, one)])
    return jnp.where(lin & one == one,
                     lax.shift_right_logical(w, s16), w & m16)

  def hist_add(hk):
    cnt, last = plsc.scan_count(hk)
    plsc.addupdate_scatter(hist_v, [hk], cnt, mask=last)

  def p1_compute(ch, k):
    def p1_pair(u, _):
      hk0 = key_vreg(2 * u, k)
      hk1 = key_vreg(2 * u + 1, k)
      keys_v[pl.ds(ch * _CH + u * 32, 32)] = plsc.pack(
          hk0, hk1, format=_ILV, preferred_element_type=jnp.int16)
      hist_add(hk0)
      hist_add(hk1)
      return 0
    lax.fori_loop(0, _PPC, p1_pair, 0)

  p1_load(0, 0)
  def p1_iter(i, _):
    ch0 = 2 * i
    p1_drain(0)
    p1_load(ch0 + 1, 1)
    p1_compute(ch0, 0)
    p1_drain(1)
    p1_load((ch0 + 2) % _NCH, 0)
    p1_compute(ch0 + 1, 1)
    return 0
  lax.fori_loop(0, _NCH // 2, p1_iter, 0)
  p1_drain(0)

  def pub_pair(i, _):
    m0 = hist_v[pl.ds(i * 32, _L)]
    m1 = hist_v[pl.ds(i * 32 + _L, _L)]
    cpk_v[pl.ds(i * _L, _L)] = plsc.bitcast(
        plsc.pack(m0, m1, format=_ILV, preferred_element_type=jnp.int16),
        jnp.int32)
    return 0
  lax.fori_loop(0, _HW // _L, pub_pair, 0)
  pltpu.sync_copy(cpk_v, perm_sh.at[pl.ds(s * _HW, _HW)])
  plsc.subcore_barrier()
  partner = s + 1 - 2 * half
  pltpu.sync_copy(perm_sh.at[pl.ds(partner * _HW, _HW)], cpk_v)
  hb = jnp.full((_L,), half, jnp.int32)

  def p2_pair(i, carry):
    p0, p1 = plsc.unpack(
        plsc.bitcast(cpk_v[pl.ds(i * _L, _L)], jnp.int16),
        format=_ILV, preferred_element_type=jnp.int32)
    p0 = p0 & m16
    p1 = p1 & m16
    m0 = hist_v[pl.ds(i * 32, _L)]
    m1 = hist_v[pl.ds(i * 32 + _L, _L)]
    tot0 = m0 + p0
    tot1 = m1 + p1
    incl0 = plsc.cumsum(tot0)
    incl1 = plsc.cumsum(tot1)
    carry1 = carry + jnp.sum(tot0)
    off0 = incl0 - tot0 + jnp.full((_L,), carry, jnp.int32) + p0 * hb
    off1 = incl1 - tot1 + jnp.full((_L,), carry1, jnp.int32) + p1 * hb
    hist_v[pl.ds(i * 32, _L)] = off0
    hist_v[pl.ds(i * 32 + _L, _L)] = off1
    return carry1 + jnp.sum(tot1)
  lax.fori_loop(0, _K // 32, p2_pair, jnp.int32(0))
  plsc.subcore_barrier()

  stage_base = jnp.full((_L,), pair * _N, jnp.int32)

  def rank_vreg(hk):
    cnt, last = plsc.scan_count(hk)
    base = plsc.load_gather(hist_v, [hk])
    plsc.addupdate_scatter(hist_v, [hk], cnt, mask=last)
    return base + cnt - one

  def p3_compute(ch, k):
    def p3_pair(u, _):
      k0, k1 = plsc.unpack(
          keys_v[pl.ds(ch * _CH + u * 32, 32)],
          format=_ILV, preferred_element_type=jnp.int32)
      v = 2 * u
      pos0 = rank_vreg(k0)
      pos1 = rank_vreg(k1)
      keys_v[pl.ds(ch * _CH + u * 32, 32)] = plsc.pack(
          pos0, pos1, format=_ILV, preferred_element_type=jnp.int16)
      g = v // (_GRP // _L)
      lane0 = (v % (_GRP // _L)) * _L
      eidx_v[k * _NG + g, pl.ds(lane0, _L)] = pos0 + stage_base
      eidx_v[k * _NG + g, pl.ds(lane0 + _L, _L)] = pos1 + stage_base
      ibase = half * _HALF + ch * _CH + v * _L
      ival_v[pl.ds(k * _CH + v * _L, _L)] = iota + jnp.full(
          (_L,), ibase, jnp.int32)
      ival_v[pl.ds(k * _CH + v * _L + _L, _L)] = iota + jnp.full(
          (_L,), ibase + _L, jnp.int32)
      return 0
    lax.fori_loop(0, _PPC, p3_pair, 0)

  def p3_fire(k):
    for g in range(_NG):
      pltpu.async_copy(ival_v.at[pl.ds(k * _CH + g * _GRP, _GRP)],
                       perm_sh.at[eidx_v.at[k * _NG + g]], gsem[k])

  def p3_drain(k):
    for g in range(_NG):
      pltpu.make_async_copy(ival_v.at[pl.ds(k * _CH + g * _GRP, _GRP)],
                            perm_sh.at[eidx_v.at[k * _NG + g]],
                            gsem[k]).wait()

  p3_compute(0, 0)
  p3_fire(0)
  p3_compute(1, 1)
  p3_fire(1)
  def p3_iter(i, _):
    ch0 = 2 * i
    p3_drain(0)
    p3_compute(ch0, 0)
    p3_fire(0)
    p3_drain(1)
    p3_compute(ch0 + 1, 1)
    p3_fire(1)
    return 0
  lax.fori_loop(1, _NCH // 2, p3_iter, 0)
  p3_drain(0)
  p3_drain(1)
  plsc.subcore_barrier()

  pltpu.sync_copy(perm_sh.at[pl.ds(pair * _N + half * _HALF, _HALF)],
                  oi_hbm.at[pl.ds(pt_base, _HALF)])
  plsc.subcore_barrier()

  def round_compute(hbm_in, ch, k):
    pltpu.sync_copy(hbm_in.at[pl.ds(pt_base + ch * _CH, _CH)],
                    xv.at[pl.ds(k * _CH, _CH)])
    def r_pair(u, _):
      pos0, pos1 = plsc.unpack(
          keys_v[pl.ds(ch * _CH + u * 32, 32)],
          format=_ILV, preferred_element_type=jnp.int32)
      v = 2 * u
      g = v // (_GRP // _L)
      lane0 = (v % (_GRP // _L)) * _L
      eidx_v[k * _NG + g, pl.ds(lane0, _L)] = (pos0 & m16) + stage_base
      eidx_v[k * _NG + g, pl.ds(lane0 + _L, _L)] = (pos1 & m16) + stage_base
      ival_v[pl.ds(k * _CH + v * _L, _L)] = plsc.bitcast(
          xv[pl.ds(k * _CH + v * _L, _L)], jnp.int32)
      ival_v[pl.ds(k * _CH + v * _L + _L, _L)] = plsc.bitcast(
          xv[pl.ds(k * _CH + v * _L + _L, _L)], jnp.int32)
      return 0
    lax.fori_loop(0, _PPC, r_pair, 0)

  for p, (hbm_in, _buf) in enumerate(planes):
    round_compute(hbm_in, 0, 0)
    p3_fire(0)
    round_compute(hbm_in, 1, 1)
    p3_fire(1)
    def r_iter(i, _, hbm_in=hbm_in):
      ch0 = 2 * i
      p3_drain(0)
      round_compute(hbm_in, ch0, 0)
      p3_fire(0)
      p3_drain(1)
      round_compute(hbm_in, ch0 + 1, 1)
      p3_fire(1)
      return 0
    lax.fori_loop(1, _NCH // 2, r_iter, 0)
    p3_drain(0)
    p3_drain(1)
    plsc.subcore_barrier()
    pltpu.sync_copy(
        perm_sh.at[pl.ds(pair * _N + half * _HALF, _HALF)],
        op_hbm.at[pl.ds(p * _B * _N + pt_base, _HALF)])
    plsc.subcore_barrier()


@jax.jit
def kernel(point_cloud, origin, radius, curve):
  bins = curve.shape[0]
  bin_interval = radius * 2.0 / bins
  par = jnp.stack([
      jnp.full((_L,), origin[0], jnp.float32),
      jnp.full((_L,), origin[1], jnp.float32),
      jnp.full((_L,), origin[2], jnp.float32),
      jnp.full((_L,), bin_interval, jnp.float32),
  ])
  cv = curve.reshape(_KW, 2).astype(jnp.int32)
  curve_packed = cv[:, 0] | (cv[:, 1] << 16)
  px = point_cloud[:, :, 0].reshape(_B * _N)
  py = point_cloud[:, :, 1].reshape(_B * _N)
  pz = point_cloud[:, :, 2].reshape(_B * _N)

  mesh = plsc.VectorSubcoreMesh(
      core_axis_name="c", subcore_axis_name="s",
      num_cores=_NC, num_subcores=_NS)
  splanes, sorted_idx_flat = pl.kernel(
      _sort_body,
      out_type=[
          jax.ShapeDtypeStruct((3 * _B * _N,), jnp.int32),
          jax.ShapeDtypeStruct((_B * _N,), jnp.int32),
      ],
      mesh=mesh,
      scratch_types=[
          pltpu.VMEM((_KW,), jnp.int32),
          pltpu.VMEM((_HALF,), jnp.int16),
          pltpu.VMEM((_K,), jnp.int32),
          pltpu.VMEM((2 * _CH,), jnp.float32),
          pltpu.VMEM((2 * _CH,), jnp.float32),
          pltpu.VMEM((2 * _CH,), jnp.float32),
          pltpu.VMEM((2 * _NG, _GRP), jnp.int32),
          pltpu.VMEM((2 * _CH,), jnp.int32),
          pltpu.VMEM((4, _L), jnp.float32),
          pltpu.VMEM_SHARED((_NS * _K,), jnp.int32),
          pltpu.SemaphoreType.DMA,
          pltpu.SemaphoreType.DMA,
          pltpu.SemaphoreType.DMA,
          pltpu.SemaphoreType.DMA,
      ],
      compiler_params=pltpu.CompilerParams(
          needs_layout_passes=False, use_tc_tiling_on_sc=False),
  )(px, py, pz, curve_packed, par)
  splanes_f = jax.lax.bitcast_convert_type(splanes, jnp.float32)
  sorted_points = (splanes_f.reshape(3, _B, _N).transpose(1, 2, 0)
                   - origin[None, None, :])
  return (sorted_points, sorted_idx_flat.reshape(_B, _N))
